# Initial kernel scaffold; baseline (speedup 1.0000x reference)
import jax
import jax.numpy as jnp
from jax.experimental import pallas as pl


def kernel(x, edge_index, edge_weight, W):
    raise NotImplementedError("write your pallas kernel here")


# speedup vs baseline: 4.5369x; 4.5369x over previous
import functools

import jax
import jax.numpy as jnp
from jax import lax
from jax.experimental import pallas as pl
from jax.experimental.pallas import tpu as pltpu
from jax.experimental.pallas import tpu_sc as plsc

N_NODES = 10000
N_EDGES = 320000
D = 128

NC = 2
NS = 16
L = 16
NW = NC * NS

EDGES_PER_TILE = N_EDGES // NW
CHUNK = 80
N_CHUNKS = EDGES_PER_TILE // CHUNK
ROWS_MAIN = 624
TAIL0 = NS * ROWS_MAIN
TAIL = N_NODES - TAIL0
ZROWS = 208

_mesh = plsc.VectorSubcoreMesh(core_axis_name="c", subcore_axis_name="s")


@functools.partial(
    pl.kernel,
    mesh=_mesh,
    out_type=jax.ShapeDtypeStruct((NC, N_NODES, D), jnp.float32),
    scratch_types=[
        pltpu.VMEM((CHUNK,), jnp.int32),
        pltpu.VMEM((CHUNK,), jnp.int32),
        pltpu.VMEM((CHUNK,), jnp.float32),
        pltpu.VMEM((CHUNK, D), jnp.float32),
        pltpu.VMEM((ZROWS, D), jnp.float32),
        pltpu.VMEM_SHARED((N_NODES, D), jnp.float32),
        pltpu.SemaphoreType.DMA,
    ],
)
def _sc_aggregate(x_hbm, src_hbm, dst_hbm, w_hbm, out_hbm,
                  src_v, dst_v, w_v, rows_v, zero_v, acc, sem):
    cid = lax.axis_index("c")
    sid = lax.axis_index("s")
    wid = cid * NS + sid

    zvec = jnp.zeros((L,), jnp.float32)

    def zrow(r, carry):
        for v in range(D // L):
            zero_v[r, pl.ds(v * L, L)] = zvec
        return carry

    lax.fori_loop(0, ZROWS, zrow, 0)
    r0 = pl.multiple_of(sid * ROWS_MAIN, 8)
    for k in range(ROWS_MAIN // ZROWS):
        pltpu.sync_copy(zero_v, acc.at[pl.ds(r0 + k * ZROWS, ZROWS), :])

    @pl.when(sid == NS - 1)
    def _zero_tail():
        pltpu.sync_copy(zero_v.at[pl.ds(0, TAIL), :],
                        acc.at[pl.ds(TAIL0, TAIL), :])

    plsc.subcore_barrier()

    ebase = wid * EDGES_PER_TILE

    def chunk_body(i, carry):
        base = ebase + i * CHUNK
        pltpu.sync_copy(src_hbm.at[pl.ds(base, CHUNK)], src_v)
        pltpu.sync_copy(dst_hbm.at[pl.ds(base, CHUNK)], dst_v)
        pltpu.sync_copy(w_hbm.at[pl.ds(base, CHUNK)], w_v)
        pltpu.async_copy(x_hbm.at[src_v], rows_v, sem).wait()

        def scale_group(g, c):
            wv = w_v[pl.ds(g * L, L)]
            for j in range(L):
                wb = jnp.broadcast_to(wv[j], (L,))
                e = g * L + j
                for v in range(D // L):
                    rows_v[e, pl.ds(v * L, L)] = (
                        rows_v[e, pl.ds(v * L, L)] * wb)
            return c

        lax.fori_loop(0, CHUNK // L, scale_group, 0)
        pltpu.sync_copy(rows_v, acc.at[dst_v], add=True)
        return carry

    lax.fori_loop(0, N_CHUNKS, chunk_body, 0)
    plsc.subcore_barrier()

    pltpu.sync_copy(acc.at[pl.ds(r0, ROWS_MAIN), :],
                    out_hbm.at[cid, pl.ds(r0, ROWS_MAIN), :])

    @pl.when(sid == NS - 1)
    def _write_tail():
        pltpu.sync_copy(acc.at[pl.ds(TAIL0, TAIL), :],
                        out_hbm.at[cid, pl.ds(TAIL0, TAIL), :])


BM = 1000


def _tc_body(p_ref, w_ref, o_ref):
    s = p_ref[0] + p_ref[1]
    o_ref[...] = jnp.maximum(
        jnp.dot(s, w_ref[...], preferred_element_type=jnp.float32), 0.0)


def _tc_combine(partials, W):
    return pl.pallas_call(
        _tc_body,
        grid=(N_NODES // BM,),
        in_specs=[
            pl.BlockSpec((NC, BM, D), lambda i: (0, i, 0)),
            pl.BlockSpec((D, D), lambda i: (0, 0)),
        ],
        out_specs=pl.BlockSpec((BM, D), lambda i: (i, 0)),
        out_shape=jax.ShapeDtypeStruct((N_NODES, D), jnp.float32),
    )(partials, W)


def kernel(x, edge_index, edge_weight, W):
    src = edge_index[1].astype(jnp.int32)
    dst = edge_index[0].astype(jnp.int32)
    partials = _sc_aggregate(x, src, dst, edge_weight)
    return _tc_combine(partials, W)


# speedup vs baseline: 8.1534x; 1.7971x over previous
import functools

import jax
import jax.numpy as jnp
from jax import lax
from jax.experimental import pallas as pl
from jax.experimental.pallas import tpu as pltpu
from jax.experimental.pallas import tpu_sc as plsc

N_NODES = 10000
N_EDGES = 320000
D = 128

NC = 2
NS = 16
L = 16
NW = NC * NS

EDGES_PER_TILE = N_EDGES // NW
CHUNK = 80
N_CHUNKS = EDGES_PER_TILE // CHUNK
N_QUADS = N_CHUNKS // 4
NBUF = 4
ROWS_MAIN = 624
TAIL0 = NS * ROWS_MAIN
TAIL = N_NODES - TAIL0
ZROWS = 48

_mesh = plsc.VectorSubcoreMesh(core_axis_name="c", subcore_axis_name="s")


@functools.partial(
    pl.kernel,
    mesh=_mesh,
    out_type=jax.ShapeDtypeStruct((NC, N_NODES, D), jnp.float32),
    scratch_types=(
        [pltpu.VMEM((2, CHUNK), jnp.int32) for _ in range(NBUF)]
        + [pltpu.VMEM((CHUNK,), jnp.float32) for _ in range(NBUF)]
        + [pltpu.VMEM((CHUNK, D), jnp.float32) for _ in range(NBUF)]
        + [pltpu.VMEM((ZROWS, D), jnp.float32),
           pltpu.VMEM_SHARED((N_NODES, D), jnp.float32)]
        + [pltpu.SemaphoreType.DMA for _ in range(2 * NBUF)]
    ),
)
def _sc_aggregate(x_hbm, eidx_hbm, ew_hbm, out_hbm,
                  e0, e1, e2, e3, w0, w1, w2, w3, r0_, r1_, r2_, r3_,
                  zero_v, acc, g0, g1, g2, g3, s0, s1, s2, s3):
    cid = lax.axis_index("c")
    sid = lax.axis_index("s")
    wid = cid * NS + sid
    ebufs = (e0, e1, e2, e3)
    wbufs = (w0, w1, w2, w3)
    rbufs = (r0_, r1_, r2_, r3_)
    gsems = (g0, g1, g2, g3)
    ssems = (s0, s1, s2, s3)

    zvec = jnp.zeros((L,), jnp.float32)

    def zrow(r, carry):
        for v in range(D // L):
            zero_v[r, pl.ds(v * L, L)] = zvec
        return carry

    lax.fori_loop(0, ZROWS, zrow, 0)
    row0 = pl.multiple_of(sid * ROWS_MAIN, 8)
    for k in range(ROWS_MAIN // ZROWS):
        pltpu.sync_copy(zero_v, acc.at[pl.ds(row0 + k * ZROWS, ZROWS), :])

    @pl.when(sid == NS - 1)
    def _zero_tail():
        pltpu.sync_copy(zero_v.at[pl.ds(0, TAIL), :],
                        acc.at[pl.ds(TAIL0, TAIL), :])

    plsc.subcore_barrier()

    def load_edata(i, b):
        pltpu.sync_copy(eidx_hbm.at[wid, i], ebufs[b])
        pltpu.sync_copy(ew_hbm.at[wid, i], wbufs[b])

    def start_gather(i, b):
        pltpu.async_copy(x_hbm.at[ebufs[b].at[0]], rbufs[b], gsems[b])

    def wait_gather(b):
        pltpu.make_async_copy(x_hbm.at[ebufs[b].at[0]], rbufs[b],
                              gsems[b]).wait()

    def start_scatter(b):
        pltpu.async_copy(rbufs[b], acc.at[ebufs[b].at[1]], ssems[b], add=True)

    def wait_scatter(b):
        pltpu.make_async_copy(rbufs[b], acc.at[ebufs[b].at[1]],
                              ssems[b]).wait()

    def scale(b):
        rb, wbuf = rbufs[b], wbufs[b]

        def grp(g, carry):
            wv = wbuf[pl.ds(g * L, L)]
            for j in range(L):
                wb = jnp.broadcast_to(wv[j], (L,))
                e = g * L + j
                for v in range(D // L):
                    rb[e, pl.ds(v * L, L)] = rb[e, pl.ds(v * L, L)] * wb
            return carry

        lax.fori_loop(0, CHUNK // L, grp, 0)

    load_edata(0, 0)
    start_gather(0, 0)
    load_edata(1, 1)
    start_gather(1, 1)

    def quad(q, carry):
        for b in range(NBUF):
            i = 4 * q + b
            nb = (b + 2) % NBUF

            @pl.when(i >= 2)
            def _free_slot():
                wait_scatter(nb)

            @pl.when(i <= N_CHUNKS - 3)
            def _prefetch():
                load_edata(i + 2, nb)
                start_gather(i + 2, nb)

            wait_gather(b)
            scale(b)
            start_scatter(b)
        return carry

    lax.fori_loop(0, N_QUADS, quad, 0)

    wait_scatter(2)
    wait_gather(0)
    scale(0)
    start_scatter(0)
    wait_scatter(3)
    wait_scatter(0)
    plsc.subcore_barrier()

    pltpu.sync_copy(acc.at[pl.ds(row0, ROWS_MAIN), :],
                    out_hbm.at[cid, pl.ds(row0, ROWS_MAIN), :])

    @pl.when(sid == NS - 1)
    def _write_tail():
        pltpu.sync_copy(acc.at[pl.ds(TAIL0, TAIL), :],
                        out_hbm.at[cid, pl.ds(TAIL0, TAIL), :])


BM = 1000


def _tc_body(p_ref, w_ref, o_ref):
    s = p_ref[0] + p_ref[1]
    o_ref[...] = jnp.maximum(
        jnp.dot(s, w_ref[...], preferred_element_type=jnp.float32), 0.0)


def _tc_combine(partials, W):
    return pl.pallas_call(
        _tc_body,
        grid=(N_NODES // BM,),
        in_specs=[
            pl.BlockSpec((NC, BM, D), lambda i: (0, i, 0)),
            pl.BlockSpec((D, D), lambda i: (0, 0)),
        ],
        out_specs=pl.BlockSpec((BM, D), lambda i: (i, 0)),
        out_shape=jax.ShapeDtypeStruct((N_NODES, D), jnp.float32),
    )(partials, W)


def kernel(x, edge_index, edge_weight, W):
    src = edge_index[1].astype(jnp.int32).reshape(NW, N_CHUNKS, CHUNK)
    dst = edge_index[0].astype(jnp.int32).reshape(NW, N_CHUNKS, CHUNK)
    eidx = jnp.stack([src, dst], axis=2)
    ew = edge_weight.reshape(NW, N_CHUNKS, CHUNK)
    partials = _sc_aggregate(x, eidx, ew)
    return _tc_combine(partials, W)


# speedup vs baseline: 11.8288x; 1.4508x over previous
import functools

import jax
import jax.numpy as jnp
from jax import lax
from jax.experimental import pallas as pl
from jax.experimental.pallas import tpu as pltpu
from jax.experimental.pallas import tpu_sc as plsc

N_NODES = 10000
N_EDGES = 320000
D = 128

NC = 2
NS = 16
L = 16
NW = NC * NS

EDGES_PER_TILE = N_EDGES // NW
CHUNK = 80
N_CHUNKS = EDGES_PER_TILE // CHUNK
NR = 4
NE = 8
ROWS_MAIN = 624
TAIL0 = NS * ROWS_MAIN
TAIL = N_NODES - TAIL0
ZROWS = 48

_mesh = plsc.VectorSubcoreMesh(core_axis_name="c", subcore_axis_name="s")


@functools.partial(
    pl.kernel,
    mesh=_mesh,
    out_type=jax.ShapeDtypeStruct((NC, N_NODES, D), jnp.float32),
    scratch_types=(
        [pltpu.VMEM((2, CHUNK), jnp.int32) for _ in range(NE)]
        + [pltpu.VMEM((1, CHUNK), jnp.float32) for _ in range(NE)]
        + [pltpu.VMEM((CHUNK, D), jnp.float32) for _ in range(NR)]
        + [pltpu.VMEM((ZROWS, D), jnp.float32),
           pltpu.VMEM_SHARED((N_NODES, D), jnp.float32)]
        + [pltpu.SemaphoreType.DMA] * (NE + 2 * NR + 1)
    ),
)
def _sc_aggregate(x_hbm, eidx_hbm, ew_hbm, out_hbm, *refs):
    ebufs = refs[0:NE]
    wbufs = refs[NE:2 * NE]
    rbufs = refs[2 * NE:2 * NE + NR]
    zero_v = refs[2 * NE + NR]
    acc = refs[2 * NE + NR + 1]
    isems = refs[2 * NE + NR + 2:3 * NE + NR + 2]
    gsems = refs[3 * NE + NR + 2:3 * NE + 2 * NR + 2]
    ssems = refs[3 * NE + 2 * NR + 2:3 * NE + 3 * NR + 2]
    zsem = refs[3 * NE + 3 * NR + 2]

    cid = lax.axis_index("c")
    sid = lax.axis_index("s")
    wid = cid * NS + sid

    def load_edata(i, b):
        eb = b % NE
        pltpu.async_copy(eidx_hbm.at[wid, i], ebufs[eb], isems[eb])
        pltpu.async_copy(ew_hbm.at[wid, pl.ds(i, 1), :], wbufs[eb],
                         isems[eb])

    def wait_edata(b):
        eb = b % NE
        pltpu.make_async_copy(eidx_hbm.at[wid, 0], ebufs[eb],
                              isems[eb]).wait()
        pltpu.make_async_copy(ew_hbm.at[wid, pl.ds(0, 1), :], wbufs[eb],
                              isems[eb]).wait()

    def start_gather(b):
        pltpu.async_copy(x_hbm.at[ebufs[b % NE].at[0]], rbufs[b % NR],
                         gsems[b % NR])

    def wait_gather(b):
        pltpu.make_async_copy(x_hbm.at[ebufs[b % NE].at[0]], rbufs[b % NR],
                              gsems[b % NR]).wait()

    def start_scatter(b):
        pltpu.async_copy(rbufs[b % NR], acc.at[ebufs[b % NE].at[1]],
                         ssems[b % NR], add=True)

    def wait_scatter(b):
        pltpu.make_async_copy(rbufs[b % NR], acc.at[ebufs[b % NE].at[1]],
                              ssems[b % NR]).wait()

    def scale(b):
        rb, wbuf = rbufs[b % NR], wbufs[b % NE]

        def grp(g, carry):
            wv = wbuf[0, pl.ds(g * L, L)]
            for j in range(L):
                wb = jnp.broadcast_to(wv[j], (L,))
                e = g * L + j
                for v in range(D // L):
                    rb[e, pl.ds(v * L, L)] = rb[e, pl.ds(v * L, L)] * wb
            return carry

        lax.fori_loop(0, CHUNK // L, grp, 0, unroll=1)

    def chunk_step(i, b, skip_free=False, skip_load=False, skip_gather=False):
        if not skip_free:
            wait_scatter(b - 2)
        if not skip_load:
            load_edata(i + 4, b + 4)
        if not skip_gather:
            wait_edata(b + 2)
            start_gather(b + 2)
        wait_gather(b)
        scale(b)
        start_scatter(b)

    for j in range(4):
        load_edata(j, j)

    zvec = jnp.zeros((L,), jnp.float32)

    def zrow(r, carry):
        for v in range(D // L):
            zero_v[r, pl.ds(v * L, L)] = zvec
        return carry

    lax.fori_loop(0, ZROWS, zrow, 0)
    row0 = pl.multiple_of(sid * ROWS_MAIN, 8)
    zcopies = [(pl.ds(row0 + k * ZROWS, ZROWS), ZROWS)
               for k in range(ROWS_MAIN // ZROWS)]
    for sl, _ in zcopies:
        pltpu.async_copy(zero_v, acc.at[sl, :], zsem)

    @pl.when(sid == NS - 1)
    def _zero_tail():
        pltpu.sync_copy(zero_v.at[pl.ds(0, TAIL), :],
                        acc.at[pl.ds(TAIL0, TAIL), :])

    wait_edata(0)
    start_gather(0)
    wait_edata(1)
    start_gather(1)
    for sl, _ in zcopies:
        pltpu.make_async_copy(zero_v, acc.at[sl, :], zsem).wait()
    plsc.subcore_barrier()

    for i in range(8):
        chunk_step(i, i, skip_free=(i < 2))

    def octet(t, carry):
        i0 = 8 * t
        for b in range(8):
            chunk_step(i0 + b, b)
        return carry

    lax.fori_loop(1, N_CHUNKS // 8, octet, 0)

    for i in range(120, N_CHUNKS):
        chunk_step(i, i, skip_load=(i + 4 > N_CHUNKS - 1),
                   skip_gather=(i + 2 > N_CHUNKS - 1))

    wait_scatter(N_CHUNKS - 2)
    wait_scatter(N_CHUNKS - 1)
    plsc.subcore_barrier()

    pltpu.sync_copy(acc.at[pl.ds(row0, ROWS_MAIN), :],
                    out_hbm.at[cid, pl.ds(row0, ROWS_MAIN), :])

    @pl.when(sid == NS - 1)
    def _write_tail():
        pltpu.sync_copy(acc.at[pl.ds(TAIL0, TAIL), :],
                        out_hbm.at[cid, pl.ds(TAIL0, TAIL), :])


BM = 1000


def _tc_body(p_ref, w_ref, o_ref):
    s = p_ref[0] + p_ref[1]
    o_ref[...] = jnp.maximum(
        jnp.dot(s, w_ref[...], preferred_element_type=jnp.float32), 0.0)


def _tc_combine(partials, W):
    return pl.pallas_call(
        _tc_body,
        grid=(N_NODES // BM,),
        in_specs=[
            pl.BlockSpec((NC, BM, D), lambda i: (0, i, 0)),
            pl.BlockSpec((D, D), lambda i: (0, 0)),
        ],
        out_specs=pl.BlockSpec((BM, D), lambda i: (i, 0)),
        out_shape=jax.ShapeDtypeStruct((N_NODES, D), jnp.float32),
    )(partials, W)


def kernel(x, edge_index, edge_weight, W):
    src = edge_index[1].astype(jnp.int32).reshape(NW, N_CHUNKS, CHUNK)
    dst = edge_index[0].astype(jnp.int32).reshape(NW, N_CHUNKS, CHUNK)
    eidx = jnp.stack([src, dst], axis=2)
    ew = edge_weight.reshape(NW, N_CHUNKS, CHUNK)
    partials = _sc_aggregate(x, eidx, ew)
    return _tc_combine(partials, W)


# speedup vs baseline: 12.6472x; 1.0692x over previous
import functools

import jax
import jax.numpy as jnp
from jax import lax
from jax.experimental import pallas as pl
from jax.experimental.pallas import tpu as pltpu
from jax.experimental.pallas import tpu_sc as plsc

N_NODES = 10000
N_EDGES = 320000
D = 128

NC = 2
NS = 16
L = 16
NW = NC * NS

EDGES_PER_TILE = N_EDGES // NW
CHUNK = 80
N_CHUNKS = EDGES_PER_TILE // CHUNK
NR = 4
NE = 8
ROWS_MAIN = 624
TAIL0 = NS * ROWS_MAIN
TAIL = N_NODES - TAIL0
ZROWS = 48

_mesh = plsc.VectorSubcoreMesh(core_axis_name="c", subcore_axis_name="s")


@functools.partial(
    pl.kernel,
    mesh=_mesh,
    out_type=jax.ShapeDtypeStruct((NC, N_NODES, D), jnp.float32),
    scratch_types=(
        [pltpu.VMEM((1, CHUNK), jnp.int32) for _ in range(NE)]
        + [pltpu.VMEM((1, CHUNK), jnp.int32) for _ in range(NE)]
        + [pltpu.VMEM((1, CHUNK), jnp.float32) for _ in range(NE)]
        + [pltpu.VMEM((CHUNK, D), jnp.float32) for _ in range(NR)]
        + [pltpu.VMEM((ZROWS, D), jnp.float32),
           pltpu.VMEM_SHARED((N_NODES, D), jnp.float32)]
        + [pltpu.SemaphoreType.DMA] * (NE + 2 * NR + 1)
    ),
)
def _sc_aggregate(x_hbm, es_hbm, ed_hbm, ew_hbm, out_hbm, *refs):
    sbufs = refs[0:NE]
    dbufs = refs[NE:2 * NE]
    wbufs = refs[2 * NE:3 * NE]
    rbufs = refs[3 * NE:3 * NE + NR]
    zero_v = refs[3 * NE + NR]
    acc = refs[3 * NE + NR + 1]
    isems = refs[3 * NE + NR + 2:4 * NE + NR + 2]
    gsems = refs[4 * NE + NR + 2:4 * NE + 2 * NR + 2]
    ssems = refs[4 * NE + 2 * NR + 2:4 * NE + 3 * NR + 2]
    zsem = refs[4 * NE + 3 * NR + 2]

    cid = lax.axis_index("c")
    sid = lax.axis_index("s")
    wid = cid * NS + sid

    def load_edata(i, b):
        eb = b % NE
        sl = pl.ds(i, 1)
        pltpu.async_copy(es_hbm.at[wid, sl, :], sbufs[eb], isems[eb])
        pltpu.async_copy(ed_hbm.at[wid, sl, :], dbufs[eb], isems[eb])
        pltpu.async_copy(ew_hbm.at[wid, sl, :], wbufs[eb], isems[eb])

    def wait_edata(b):
        eb = b % NE
        sl = pl.ds(0, 1)
        pltpu.make_async_copy(es_hbm.at[wid, sl, :], sbufs[eb],
                              isems[eb]).wait()
        pltpu.make_async_copy(ed_hbm.at[wid, sl, :], dbufs[eb],
                              isems[eb]).wait()
        pltpu.make_async_copy(ew_hbm.at[wid, sl, :], wbufs[eb],
                              isems[eb]).wait()

    def start_gather(b):
        pltpu.async_copy(x_hbm.at[sbufs[b % NE].at[0]], rbufs[b % NR],
                         gsems[b % NR])

    def wait_gather(b):
        pltpu.make_async_copy(x_hbm.at[sbufs[b % NE].at[0]], rbufs[b % NR],
                              gsems[b % NR]).wait()

    def start_scatter(b):
        pltpu.async_copy(rbufs[b % NR], acc.at[dbufs[b % NE].at[0]],
                         ssems[b % NR], add=True)

    def wait_scatter(b):
        pltpu.make_async_copy(rbufs[b % NR], acc.at[dbufs[b % NE].at[0]],
                              ssems[b % NR]).wait()

    def scale(b):
        rb, wbuf = rbufs[b % NR], wbufs[b % NE]

        def grp(g, carry):
            wv = wbuf[0, pl.ds(g * L, L)]
            for j in range(L):
                wb = jnp.broadcast_to(wv[j], (L,))
                e = g * L + j
                for v in range(D // L):
                    rb[e, pl.ds(v * L, L)] = rb[e, pl.ds(v * L, L)] * wb
            return carry

        lax.fori_loop(0, CHUNK // L, grp, 0, unroll=1)

    def chunk_step(i, b, skip_free=False, skip_load=False, skip_gather=False):
        if not skip_free:
            wait_scatter(b - 2)
        if not skip_load:
            load_edata(i + 4, b + 4)
        if not skip_gather:
            wait_edata(b + 2)
            start_gather(b + 2)
        wait_gather(b)
        scale(b)
        start_scatter(b)

    for j in range(4):
        load_edata(j, j)

    zvec = jnp.zeros((L,), jnp.float32)

    def zrow(r, carry):
        for v in range(D // L):
            zero_v[r, pl.ds(v * L, L)] = zvec
        return carry

    lax.fori_loop(0, ZROWS, zrow, 0)
    row0 = pl.multiple_of(sid * ROWS_MAIN, 8)
    zcopies = [(pl.ds(row0 + k * ZROWS, ZROWS), ZROWS)
               for k in range(ROWS_MAIN // ZROWS)]
    for sl, _ in zcopies:
        pltpu.async_copy(zero_v, acc.at[sl, :], zsem)

    @pl.when(sid == NS - 1)
    def _zero_tail():
        pltpu.sync_copy(zero_v.at[pl.ds(0, TAIL), :],
                        acc.at[pl.ds(TAIL0, TAIL), :])

    wait_edata(0)
    start_gather(0)
    wait_edata(1)
    start_gather(1)
    for sl, _ in zcopies:
        pltpu.make_async_copy(zero_v, acc.at[sl, :], zsem).wait()
    plsc.subcore_barrier()

    for i in range(8):
        chunk_step(i, i, skip_free=(i < 2))

    def octet(t, carry):
        i0 = 8 * t
        for b in range(8):
            chunk_step(i0 + b, b)
        return carry

    lax.fori_loop(1, N_CHUNKS // 8, octet, 0)

    for i in range(120, N_CHUNKS):
        chunk_step(i, i, skip_load=(i + 4 > N_CHUNKS - 1),
                   skip_gather=(i + 2 > N_CHUNKS - 1))

    wait_scatter(N_CHUNKS - 2)
    wait_scatter(N_CHUNKS - 1)
    plsc.subcore_barrier()

    pltpu.sync_copy(acc.at[pl.ds(row0, ROWS_MAIN), :],
                    out_hbm.at[cid, pl.ds(row0, ROWS_MAIN), :])

    @pl.when(sid == NS - 1)
    def _write_tail():
        pltpu.sync_copy(acc.at[pl.ds(TAIL0, TAIL), :],
                        out_hbm.at[cid, pl.ds(TAIL0, TAIL), :])


BM = 1000


def _tc_body(p_ref, w_ref, o_ref):
    s = p_ref[0] + p_ref[1]
    o_ref[...] = jnp.maximum(
        jnp.dot(s, w_ref[...], preferred_element_type=jnp.float32), 0.0)


def _tc_combine(partials, W):
    return pl.pallas_call(
        _tc_body,
        grid=(N_NODES // BM,),
        in_specs=[
            pl.BlockSpec((NC, BM, D), lambda i: (0, i, 0)),
            pl.BlockSpec((D, D), lambda i: (0, 0)),
        ],
        out_specs=pl.BlockSpec((BM, D), lambda i: (i, 0)),
        out_shape=jax.ShapeDtypeStruct((N_NODES, D), jnp.float32),
    )(partials, W)


def kernel(x, edge_index, edge_weight, W):
    src = edge_index[1].astype(jnp.int32).reshape(NW, N_CHUNKS, CHUNK)
    dst = edge_index[0].astype(jnp.int32).reshape(NW, N_CHUNKS, CHUNK)
    ew = edge_weight.reshape(NW, N_CHUNKS, CHUNK)
    partials = _sc_aggregate(x, src, dst, ew)
    return _tc_combine(partials, W)
